# baseline (speedup 1.0000x reference)
import jax
import jax.numpy as jnp
from jax import lax
from jax.experimental import pallas as pl
from jax.experimental.pallas import tpu as pltpu
from jax.experimental.pallas import tpu_sc as plsc

N_USERS = 1000000
N_ITEMS = 1000000
EMBED_DIM = 32
BATCH = 16384

NC = 2
NS = 16
L = 16
NW = NC * NS
B_PER_W = BATCH // NW
IDX_CHUNK = 128
N_CHUNKS = B_PER_W // IDX_CHUNK
G_PER_CHUNK = IDX_CHUNK // L


def _mf_kernel(user_idx_hbm, item_idx_hbm, user_table, item_table,
               user_bias, item_bias, gb_hbm, out_hbm,
               idx_u, idx_i, el_u, el_i, cols_u, cols_i,
               bias_u, bias_i, gb_v, out_v, sem, sem_b):
    wid = lax.axis_index("s") * NC + lax.axis_index("c")
    base_blk = wid * N_CHUNKS

    pltpu.sync_copy(user_idx_hbm.at[pl.ds(base_blk, N_CHUNKS)], idx_u)
    pltpu.sync_copy(item_idx_hbm.at[pl.ds(base_blk, N_CHUNKS)], idx_i)
    pltpu.sync_copy(gb_hbm, gb_v)

    bias_copies = []
    for j in range(N_CHUNKS):
        s = pl.ds(j * IDX_CHUNK, IDX_CHUNK)
        bias_copies.append(pltpu.async_copy(
            user_bias.at[idx_u.at[j]], bias_u.at[s], sem_b))
        bias_copies.append(pltpu.async_copy(
            item_bias.at[idx_i.at[j]], bias_i.at[s], sem_b))

    def build(g, _):
        j = g // G_PER_CHUNK
        s = pl.ds((g % G_PER_CHUNK) * L, L)
        bu = idx_u[j, s] << 5
        bi = idx_i[j, s] << 5
        for d in range(EMBED_DIM):
            el_u[d, j, s] = bu + d
            el_i[d, j, s] = bi + d
        return _

    lax.fori_loop(0, N_CHUNKS * G_PER_CHUNK, build, None)

    for c in bias_copies:
        c.wait()
    gb = gb_v[...]

    for j in range(N_CHUNKS):
        copies = []
        for d in range(EMBED_DIM):
            copies.append(pltpu.async_copy(
                user_table.at[el_u.at[d, j]], cols_u.at[d], sem))
            copies.append(pltpu.async_copy(
                item_table.at[el_i.at[d, j]], cols_i.at[d], sem))
        for c in copies:
            c.wait()

        def body(g, _, j=j):
            s = pl.ds(g * L, L)
            acc = None
            for d in range(EMBED_DIM):
                prod = cols_u[d, s] * cols_i[d, s]
                acc = prod if acc is None else acc + prod
            so = pl.ds(j * IDX_CHUNK + g * L, L)
            p = acc + bias_u[so] + bias_i[so] + gb
            out_v[so] = 1.0 / (1.0 + jnp.exp(-p))
            return _

        lax.fori_loop(0, G_PER_CHUNK, body, None)

    pltpu.sync_copy(out_v, out_hbm.at[pl.ds(wid * B_PER_W, B_PER_W)])


def kernel(user_idx, item_idx, user_table, item_table, user_bias_table,
           item_bias_table, global_bias):
    mesh = plsc.VectorSubcoreMesh(core_axis_name="c", subcore_axis_name="s")
    run = pl.kernel(
        _mf_kernel,
        mesh=mesh,
        compiler_params=pltpu.CompilerParams(needs_layout_passes=False),
        out_type=jax.ShapeDtypeStruct((BATCH,), jnp.float32),
        scratch_types=[
            pltpu.VMEM((N_CHUNKS, IDX_CHUNK), jnp.int32),
            pltpu.VMEM((N_CHUNKS, IDX_CHUNK), jnp.int32),
            pltpu.VMEM((EMBED_DIM, N_CHUNKS, IDX_CHUNK), jnp.int32),
            pltpu.VMEM((EMBED_DIM, N_CHUNKS, IDX_CHUNK), jnp.int32),
            pltpu.VMEM((EMBED_DIM, IDX_CHUNK), jnp.float32),
            pltpu.VMEM((EMBED_DIM, IDX_CHUNK), jnp.float32),
            pltpu.VMEM((B_PER_W,), jnp.float32),
            pltpu.VMEM((B_PER_W,), jnp.float32),
            pltpu.VMEM((L,), jnp.float32),
            pltpu.VMEM((B_PER_W,), jnp.float32),
            pltpu.SemaphoreType.DMA,
            pltpu.SemaphoreType.DMA,
        ],
    )
    uidx = user_idx.astype(jnp.int32).reshape(BATCH // IDX_CHUNK, IDX_CHUNK)
    iidx = item_idx.astype(jnp.int32).reshape(BATCH // IDX_CHUNK, IDX_CHUNK)
    gb16 = jnp.broadcast_to(global_bias.astype(jnp.float32), (L,))
    return run(uidx, iidx,
               user_table.reshape(N_USERS * EMBED_DIM),
               item_table.reshape(N_ITEMS * EMBED_DIM),
               user_bias_table.reshape(N_USERS),
               item_bias_table.reshape(N_ITEMS),
               gb16)


# speedup vs baseline: 1.3638x; 1.3638x over previous
import jax
import jax.numpy as jnp
from jax import lax
from jax.experimental import pallas as pl
from jax.experimental.pallas import tpu as pltpu
from jax.experimental.pallas import tpu_sc as plsc

N_USERS = 1000000
N_ITEMS = 1000000
EMBED_DIM = 32
BATCH = 16384

NC = 2
NS = 16
L = 16
NW = NC * NS
B_PER_W = BATCH // NW
GROUP = 32
N_GROUPS = B_PER_W // GROUP
VPG = GROUP // L
HALF = EMBED_DIM // 2


def _mf_kernel(user_idx_hbm, item_idx_hbm, user_table, item_table,
               user_bias, item_bias, gb_hbm, out_hbm,
               idx_u, idx_i, rows_u, rows_i,
               bias_u, bias_i, gb_v, tbuf, out_v, sem_u, sem_i, sem_b):
    wid = lax.axis_index("s") * NC + lax.axis_index("c")
    base = wid * B_PER_W

    pltpu.sync_copy(user_idx_hbm.at[pl.ds(base, B_PER_W)], idx_u)
    pltpu.sync_copy(item_idx_hbm.at[pl.ds(base, B_PER_W)], idx_i)
    pltpu.sync_copy(gb_hbm, gb_v)

    bu = pltpu.async_copy(user_bias.at[idx_u], bias_u, sem_b)
    bi = pltpu.async_copy(item_bias.at[idx_i], bias_i, sem_b)
    bu.wait()
    bi.wait()
    gb = gb_v[...]

    lane = lax.iota(jnp.int32, L)
    scatter_idx = [lane * L + b for b in range(L)]

    def fire(g):
        slot = (g & 1) * GROUP
        for v in range(VPG):
            vu = idx_u[pl.ds(g * GROUP + v * L, L)]
            vi = idx_i[pl.ds(g * GROUP + v * L, L)]
            for b in range(L):
                r_u = vu[b]
                r_i = vi[b]
                pltpu.async_copy(user_table.at[pl.ds(r_u, 1)],
                                 rows_u.at[pl.ds(slot + v * L + b, 1)], sem_u)
                pltpu.async_copy(item_table.at[pl.ds(r_i, 1)],
                                 rows_i.at[pl.ds(slot + v * L + b, 1)], sem_i)

    def drain():
        pltpu.make_async_copy(user_table.at[pl.ds(0, GROUP)],
                              rows_u.at[pl.ds(0, GROUP)], sem_u).wait()
        pltpu.make_async_copy(item_table.at[pl.ds(0, GROUP)],
                              rows_i.at[pl.ds(0, GROUP)], sem_i).wait()

    def compute(g):
        slot = (g & 1) * GROUP
        for v in range(VPG):
            for b in range(L):
                r = slot + v * L + b
                u0 = rows_u[r, pl.ds(0, HALF)]
                u1 = rows_u[r, pl.ds(HALF, HALF)]
                i0 = rows_i[r, pl.ds(0, HALF)]
                i1 = rows_i[r, pl.ds(HALF, HALF)]
                plsc.store_scatter(tbuf, [scatter_idx[b]], u0 * i0 + u1 * i1)
            acc = tbuf[pl.ds(0, L)]
            for l in range(1, L):
                acc = acc + tbuf[pl.ds(l * L, L)]
            s = pl.ds(g * GROUP + v * L, L)
            p = acc + bias_u[s] + bias_i[s] + gb
            out_v[s] = 1.0 / (1.0 + jnp.exp(-p))

    def body(g, _):
        fire(g)

        @pl.when(g > 0)
        def _prev():
            drain()
            compute(g - 1)

        return _

    lax.fori_loop(0, N_GROUPS, body, None)
    drain()
    compute(N_GROUPS - 1)

    pltpu.sync_copy(out_v, out_hbm.at[pl.ds(base, B_PER_W)])


def kernel(user_idx, item_idx, user_table, item_table, user_bias_table,
           item_bias_table, global_bias):
    mesh = plsc.VectorSubcoreMesh(core_axis_name="c", subcore_axis_name="s")
    run = pl.kernel(
        _mf_kernel,
        mesh=mesh,
        compiler_params=pltpu.CompilerParams(needs_layout_passes=False),
        out_type=jax.ShapeDtypeStruct((BATCH,), jnp.float32),
        scratch_types=[
            pltpu.VMEM((B_PER_W,), jnp.int32),
            pltpu.VMEM((B_PER_W,), jnp.int32),
            pltpu.VMEM((2 * GROUP, EMBED_DIM), jnp.float32),
            pltpu.VMEM((2 * GROUP, EMBED_DIM), jnp.float32),
            pltpu.VMEM((B_PER_W,), jnp.float32),
            pltpu.VMEM((B_PER_W,), jnp.float32),
            pltpu.VMEM((L,), jnp.float32),
            pltpu.VMEM((L * L,), jnp.float32),
            pltpu.VMEM((B_PER_W,), jnp.float32),
            pltpu.SemaphoreType.DMA,
            pltpu.SemaphoreType.DMA,
            pltpu.SemaphoreType.DMA,
        ],
    )
    gb16 = jnp.broadcast_to(global_bias.astype(jnp.float32), (L,))
    return run(user_idx.astype(jnp.int32), item_idx.astype(jnp.int32),
               user_table, item_table,
               user_bias_table.reshape(N_USERS),
               item_bias_table.reshape(N_ITEMS),
               gb16)
